# baseline (speedup 1.0000x reference)
import functools

import jax
import jax.numpy as jnp
from jax import lax
from jax.experimental import pallas as pl
from jax.experimental.pallas import tpu as pltpu
from jax.experimental.pallas import tpu_sc as plsc

N = 4096
E = 2048
D = 256
K128 = 128

_SCC = 2
_SCT = 16
_NW = _SCC * _SCT
_EPW = E // _NW
_RPT = N // _SCT


def _sc_gather(nodes, idx):
    mesh = plsc.VectorSubcoreMesh(core_axis_name="c", subcore_axis_name="s")

    @functools.partial(
        pl.kernel,
        out_type=jax.ShapeDtypeStruct((E, D), jnp.float32),
        mesh=mesh,
        scratch_types=[
            pltpu.VMEM((_EPW,), jnp.int32),
            pltpu.VMEM((_EPW, D), jnp.float32),
            pltpu.SemaphoreType.DMA,
        ],
    )
    def k(nodes_hbm, idx_hbm, out_hbm, idx_v, rows_v, sem):
        wid = lax.axis_index("s") * _SCC + lax.axis_index("c")
        base = wid * _EPW
        pltpu.sync_copy(idx_hbm.at[pl.ds(base, _EPW)], idx_v)
        pltpu.async_copy(nodes_hbm.at[idx_v], rows_v, sem).wait()
        pltpu.sync_copy(rows_v, out_hbm.at[pl.ds(base, _EPW)])

    return k(nodes, idx)


NB = 512


def _scatter_body(colrow_ref, msg_ref, agg_ref):
    i = pl.program_id(0)
    ni = lax.broadcasted_iota(jnp.int32, (NB, E), 0) + i * NB
    oh = (colrow_ref[...] == ni).astype(jnp.float32)
    agg_ref[...] = jnp.dot(oh, msg_ref[...], preferred_element_type=jnp.float32,
                           precision=_EXACT)


def _scatter(colrow, msg):
    return pl.pallas_call(
        _scatter_body,
        grid=(N // NB,),
        in_specs=[
            pl.BlockSpec((1, E), lambda i: (0, 0)),
            pl.BlockSpec((E, D), lambda i: (0, 0)),
        ],
        out_specs=pl.BlockSpec((NB, D), lambda i: (i, 0)),
        out_shape=jax.ShapeDtypeStruct((N, D), jnp.float32),
    )(colrow, msg)

_DEF = jax.lax.Precision.DEFAULT
_EXACT = jax.lax.Precision.HIGHEST


def _prologue_body(x_ref, pos_ref, col16_ref, table_ref, w0t_ref, b0_ref,
                   ea_ref, m1wt_ref, m1b_ref, out0_ref, hid_ref, invdeg_ref):
    xv = x_ref[...]
    oh = (xv == lax.broadcasted_iota(jnp.int32, (N, 5), 1)).astype(jnp.float32)
    emb = jnp.dot(oh, table_ref[...], preferred_element_type=jnp.float32,
                  precision=_EXACT)
    in_p = jnp.concatenate([emb, pos_ref[...]], axis=1)
    out0 = jnp.dot(in_p, w0t_ref[...], preferred_element_type=jnp.float32,
                   precision=_DEF)
    out0_ref[...] = jnp.maximum(out0 + b0_ref[...], 0.0)

    hid = jnp.dot(ea_ref[...], m1wt_ref[...], preferred_element_type=jnp.float32,
                  precision=_DEF)
    hid_ref[...] = jnp.maximum(hid + m1b_ref[...], 0.0)

    node_iota = lax.broadcasted_iota(jnp.int32, (N, 128), 0)
    acc = jnp.zeros((N, 1), jnp.float32)
    for c in range(E // 128):
        cmp = (node_iota == col16_ref[c:c + 1, :]).astype(jnp.float32)
        acc = acc + jnp.sum(cmp, axis=1, keepdims=True)
    deg = jnp.maximum(acc, 1.0)
    invdeg_ref[...] = 1.0 / deg


def _prologue(x, pos, col16, table, w0t, b0, ea, m1wt, m1b):
    return pl.pallas_call(
        _prologue_body,
        out_shape=(
            jax.ShapeDtypeStruct((N, D), jnp.float32),
            jax.ShapeDtypeStruct((E, K128), jnp.float32),
            jax.ShapeDtypeStruct((N, 1), jnp.float32),
        ),
    )(x, pos, col16, table, w0t, b0, ea, m1wt, m1b)


BE = 256
NC = 512
DG = 8


def _msg_body(src_ref, hid_ref, m2t_ref, b2_ref, msg_ref):
    src = src_ref[...].astype(jnp.bfloat16).astype(jnp.float32)
    hidb = hid_ref[...].astype(jnp.bfloat16)

    msg = jnp.zeros((BE, D), jnp.float32)
    for i in range(D // DG):
        wg = jnp.dot(hidb, m2t_ref[:, i * DG * D:(i + 1) * DG * D],
                     preferred_element_type=jnp.float32, precision=_DEF)
        wg = wg + b2_ref[0:1, i * DG * D:(i + 1) * DG * D]
        wg = wg.astype(jnp.bfloat16).astype(jnp.float32)
        for j in range(DG):
            d = i * DG + j
            msg = msg + src[:, d:d + 1] * wg[:, j * D:(j + 1) * D]
    msg_ref[...] = msg


def _msg(src, hid, m2t_bf16, b2row):
    return pl.pallas_call(
        _msg_body,
        grid=(E // BE,),
        in_specs=[
            pl.BlockSpec((BE, D), lambda i: (i, 0)),
            pl.BlockSpec((BE, K128), lambda i: (i, 0)),
            pl.BlockSpec((K128, D * D), lambda i: (0, 0)),
            pl.BlockSpec((1, D * D), lambda i: (0, 0)),
        ],
        out_specs=pl.BlockSpec((BE, D), lambda i: (i, 0)),
        out_shape=jax.ShapeDtypeStruct((E, D), jnp.float32),
    )(src, hid, m2t_bf16, b2row)


GB = 1024


def _gru_body(agg_ref, invdeg_ref, h_ref, convb_ref, wiht_ref, whht_ref,
              bih_ref, bhh_ref, hnew_ref):
    m = jnp.maximum(agg_ref[...] * invdeg_ref[...] + convb_ref[...], 0.0)
    gi = jnp.dot(m, wiht_ref[...], preferred_element_type=jnp.float32,
                 precision=_DEF) + bih_ref[...]
    h = h_ref[...]
    gh = jnp.dot(h, whht_ref[...], preferred_element_type=jnp.float32,
                 precision=_DEF) + bhh_ref[...]
    r = jax.nn.sigmoid(gi[:, :D] + gh[:, :D])
    z = jax.nn.sigmoid(gi[:, D:2 * D] + gh[:, D:2 * D])
    n = jnp.tanh(gi[:, 2 * D:] + r * gh[:, 2 * D:])
    hnew_ref[...] = (1.0 - z) * n + z * h


def _gru(agg, invdeg, h, convb, wiht, whht, bih, bhh):
    return pl.pallas_call(
        _gru_body,
        grid=(N // GB,),
        in_specs=[
            pl.BlockSpec((GB, D), lambda i: (i, 0)),
            pl.BlockSpec((GB, 1), lambda i: (i, 0)),
            pl.BlockSpec((GB, D), lambda i: (i, 0)),
            pl.BlockSpec((1, D), lambda i: (0, 0)),
            pl.BlockSpec((D, 3 * D), lambda i: (0, 0)),
            pl.BlockSpec((D, 3 * D), lambda i: (0, 0)),
            pl.BlockSpec((1, 3 * D), lambda i: (0, 0)),
            pl.BlockSpec((1, 3 * D), lambda i: (0, 0)),
        ],
        out_specs=pl.BlockSpec((GB, D), lambda i: (i, 0)),
        out_shape=jax.ShapeDtypeStruct((N, D), jnp.float32),
    )(agg, invdeg, h, convb, wiht, whht, bih, bhh)


def _readout_body(srcr_ref, srcc_ref, w1rt_ref, w1ct_ref, b1_ref,
                  w2t_ref, b2_ref, w6rt_ref, w6ct_ref, b6a_ref, bng_ref,
                  bnb_ref, w6bt_ref, b6b_ref, o2_ref, precs_ref):
    accr = srcr_ref[...]
    accc = srcc_ref[...]
    o1 = jnp.maximum(
        jnp.dot(accr, w1rt_ref[...], preferred_element_type=jnp.float32,
                precision=_DEF)
        + jnp.dot(accc, w1ct_ref[...], preferred_element_type=jnp.float32,
                  precision=_DEF)
        + b1_ref[...], 0.0)
    o2_ref[...] = jnp.dot(o1, w2t_ref[...], preferred_element_type=jnp.float32,
                          precision=_DEF) + b2_ref[...]
    p = (jnp.dot(accr, w6rt_ref[...], preferred_element_type=jnp.float32,
                 precision=_DEF)
         + jnp.dot(accc, w6ct_ref[...], preferred_element_type=jnp.float32,
                   precision=_DEF)
         + b6a_ref[...])
    mu = jnp.mean(p, axis=0, keepdims=True)
    var = jnp.mean((p - mu) ** 2, axis=0, keepdims=True)
    p = (p - mu) * lax.rsqrt(var + 1e-5) * bng_ref[...] + bnb_ref[...]
    p = jnp.maximum(p, 0.0)
    precs_ref[...] = jnp.dot(p, w6bt_ref[...], preferred_element_type=jnp.float32,
                             precision=_DEF) + b6b_ref[...]


def _readout(srcr, srcc, w1rt, w1ct, b1, w2t, b2, w6rt, w6ct, b6a,
             bng, bnb, w6bt, b6b):
    return pl.pallas_call(
        _readout_body,
        out_shape=(
            jax.ShapeDtypeStruct((E, D), jnp.float32),
            jax.ShapeDtypeStruct((E, 128), jnp.float32),
        ),
    )(srcr, srcc, w1rt, w1ct, b1, w2t, b2, w6rt, w6ct, b6a, bng,
      bnb, w6bt, b6b)


def kernel(x, pos, edge_index, edge_atr, node_emb_table, lin0_W, lin0_b,
           mnn1_W, mnn1_b, mnn2_W, mnn2_b, conv_b, gru_Wih, gru_Whh, gru_bih,
           gru_bhh, lin1_W, lin1_b, lin2_W, lin2_b, lin6a_W, lin6a_b, bn_g,
           bn_b, lin6b_W, lin6b_b):
    row = edge_index[0].astype(jnp.int32)
    col = edge_index[1].astype(jnp.int32)
    col16 = col.reshape(E // 128, 128)
    colrow = col.reshape(1, E)

    w0t = lin0_W.T
    b0 = lin0_b.reshape(1, D)
    m1wt = mnn1_W.T
    m1b = mnn1_b.reshape(1, K128)
    m2t = mnn2_W.T.astype(jnp.bfloat16)
    b2row = mnn2_b.reshape(1, D * D)
    convb = conv_b.reshape(1, D)
    wiht = gru_Wih.T
    whht = gru_Whh.T
    bih = gru_bih.reshape(1, 3 * D)
    bhh = gru_bhh.reshape(1, 3 * D)
    w1rt = lin1_W[:, :D].T
    w1ct = lin1_W[:, D:].T
    b1 = lin1_b.reshape(1, D)
    w2t = jnp.pad(lin2_W.T, ((0, 0), (0, D - 242)))
    b2 = jnp.pad(lin2_b, (0, D - 242)).reshape(1, D)
    w6rt = lin6a_W[:, :D].T
    w6ct = lin6a_W[:, D:].T
    b6a = lin6a_b.reshape(1, 128)
    bng = bn_g.reshape(1, 128)
    bnb = bn_b.reshape(1, 128)
    w6bt = jnp.pad(lin6b_W.T, ((0, 0), (0, 127)))
    b6b = jnp.pad(lin6b_b, (0, 127)).reshape(1, 128)

    out0, hid, invdeg = _prologue(x.astype(jnp.int32), pos, col16,
                                  node_emb_table, w0t, b0, edge_atr, m1wt, m1b)
    h = out0
    for _ in range(3):
        src = _sc_gather(h, row)
        msg = _msg(src, hid, m2t, b2row)
        agg = _scatter(colrow, msg)
        h = _gru(agg, invdeg, h, convb, wiht, whht, bih, bhh)

    src_r = _sc_gather(h, row)
    src_c = _sc_gather(h, col)
    o2p, precsp = _readout(src_r, src_c, w1rt, w1ct, b1, w2t, b2, w6rt,
                           w6ct, b6a, bng, bnb, w6bt, b6b)
    return (o2p[:, :242], precsp[:, :1])


# speedup vs baseline: 1.1767x; 1.1767x over previous
import functools

import jax
import jax.numpy as jnp
from jax import lax
from jax.experimental import pallas as pl
from jax.experimental.pallas import tpu as pltpu
from jax.experimental.pallas import tpu_sc as plsc

N = 4096
E = 2048
D = 256
K128 = 128

_SCC = 2
_SCT = 16
_NW = _SCC * _SCT
_EPW = E // _NW
_RPT = N // _SCT


def _sc_gather(nodes, idx, nrows):
    per_w = nrows // _NW
    mesh = plsc.VectorSubcoreMesh(core_axis_name="c", subcore_axis_name="s")

    @functools.partial(
        pl.kernel,
        out_type=jax.ShapeDtypeStruct((nrows, D), jnp.float32),
        mesh=mesh,
        scratch_types=[
            pltpu.VMEM((per_w,), jnp.int32),
            pltpu.VMEM((per_w, D), jnp.float32),
            pltpu.SemaphoreType.DMA,
        ],
    )
    def k(nodes_hbm, idx_hbm, out_hbm, idx_v, rows_v, sem):
        wid = lax.axis_index("s") * _SCC + lax.axis_index("c")
        base = wid * per_w
        pltpu.sync_copy(idx_hbm.at[pl.ds(base, per_w)], idx_v)
        pltpu.async_copy(nodes_hbm.at[idx_v], rows_v, sem).wait()
        pltpu.sync_copy(rows_v, out_hbm.at[pl.ds(base, per_w)])

    return k(nodes, idx)


_DEF = jax.lax.Precision.DEFAULT
_EXACT = jax.lax.Precision.HIGHEST


def _prologue_body(x_ref, pos_ref, col16_ref, table_ref, w0t_ref, b0_ref,
                   ea_ref, m1wt_ref, m1b_ref, out0_ref, hid_ref, invdeg_ref,
                   out016_ref):
    xv = x_ref[...]
    oh = (xv == lax.broadcasted_iota(jnp.int32, (N, 5), 1)).astype(jnp.float32)
    emb = jnp.dot(oh, table_ref[...], preferred_element_type=jnp.float32,
                  precision=_EXACT)
    in_p = jnp.concatenate([emb, pos_ref[...]], axis=1)
    out0 = jnp.dot(in_p, w0t_ref[...], preferred_element_type=jnp.float32,
                   precision=_DEF)
    out0 = jnp.maximum(out0 + b0_ref[...], 0.0)
    out0_ref[...] = out0
    out016_ref[...] = out0.astype(jnp.bfloat16)

    hid = jnp.dot(ea_ref[...], m1wt_ref[...], preferred_element_type=jnp.float32,
                  precision=_DEF)
    hid_ref[...] = jnp.maximum(hid + m1b_ref[...], 0.0)

    node_iota = lax.broadcasted_iota(jnp.int32, (N, 128), 0)
    acc = jnp.zeros((N, 1), jnp.float32)
    for c in range(E // 128):
        cmp = (node_iota == col16_ref[c:c + 1, :]).astype(jnp.float32)
        acc = acc + jnp.sum(cmp, axis=1, keepdims=True)
    deg = jnp.maximum(acc, 1.0)
    invdeg_ref[...] = 1.0 / deg


def _prologue(x, pos, col16, table, w0t, b0, ea, m1wt, m1b):
    return pl.pallas_call(
        _prologue_body,
        out_shape=(
            jax.ShapeDtypeStruct((N, D), jnp.float32),
            jax.ShapeDtypeStruct((E, K128), jnp.float32),
            jax.ShapeDtypeStruct((N, 1), jnp.float32),
            jax.ShapeDtypeStruct((N, D), jnp.bfloat16),
        ),
    )(x, pos, col16, table, w0t, b0, ea, m1wt, m1b)


BE = 256
NC = 512
DG = 8


def _msg_body(row_ref, hid_ref, nodes_ref, m2t_ref, b2_ref, msg_ref):
    rowb = row_ref[...]
    acc = jnp.zeros((BE, D), jnp.float32)
    for c in range(N // NC):
        ni = lax.broadcasted_iota(jnp.int32, (BE, NC), 1) + c * NC
        oh = (rowb == ni).astype(jnp.bfloat16)
        acc = acc + jnp.dot(oh, nodes_ref[c * NC:(c + 1) * NC, :],
                            preferred_element_type=jnp.float32, precision=_DEF)
    src = acc
    hidb = hid_ref[...].astype(jnp.bfloat16)

    msg = jnp.zeros((BE, D), jnp.float32)
    for i in range(D // DG):
        wg = jnp.dot(hidb, m2t_ref[:, i * DG * D:(i + 1) * DG * D],
                     preferred_element_type=jnp.float32, precision=_DEF)
        wg = wg + b2_ref[0:1, i * DG * D:(i + 1) * DG * D]
        wg = wg.astype(jnp.bfloat16).astype(jnp.float32)
        for j in range(DG):
            d = i * DG + j
            msg = msg + src[:, d:d + 1] * wg[:, j * D:(j + 1) * D]
    msg_ref[...] = msg


def _msg(row2d, hid, nodes16, m2t_bf16, b2row):
    return pl.pallas_call(
        _msg_body,
        grid=(E // BE,),
        in_specs=[
            pl.BlockSpec((BE, 1), lambda i: (i, 0)),
            pl.BlockSpec((BE, K128), lambda i: (i, 0)),
            pl.BlockSpec((N, D), lambda i: (0, 0)),
            pl.BlockSpec((K128, D * D), lambda i: (0, 0)),
            pl.BlockSpec((1, D * D), lambda i: (0, 0)),
        ],
        out_specs=pl.BlockSpec((BE, D), lambda i: (i, 0)),
        out_shape=jax.ShapeDtypeStruct((E, D), jnp.float32),
    )(row2d, hid, nodes16, m2t_bf16, b2row)


NB = 512


def _scatgru_body(colrow_ref, msg_ref, invdeg_ref, h_ref, convb_ref,
                  wiht_ref, whht_ref, bih_ref, bhh_ref, hnew_ref, hnew16_ref):
    i = pl.program_id(0)
    msg = msg_ref[...]
    hi = msg.astype(jnp.bfloat16)
    lo = (msg - hi.astype(jnp.float32)).astype(jnp.bfloat16)
    ni = lax.broadcasted_iota(jnp.int32, (NB, E), 0) + i * NB
    oh = (colrow_ref[...] == ni).astype(jnp.bfloat16)
    agg = (jnp.dot(oh, hi, preferred_element_type=jnp.float32, precision=_DEF)
           + jnp.dot(oh, lo, preferred_element_type=jnp.float32, precision=_DEF))
    m = jnp.maximum(agg * invdeg_ref[...] + convb_ref[...], 0.0)
    gi = jnp.dot(m, wiht_ref[...], preferred_element_type=jnp.float32,
                 precision=_DEF) + bih_ref[...]
    h = h_ref[...]
    gh = jnp.dot(h, whht_ref[...], preferred_element_type=jnp.float32,
                 precision=_DEF) + bhh_ref[...]
    r = jax.nn.sigmoid(gi[:, :D] + gh[:, :D])
    z = jax.nn.sigmoid(gi[:, D:2 * D] + gh[:, D:2 * D])
    n = jnp.tanh(gi[:, 2 * D:] + r * gh[:, 2 * D:])
    hnew = (1.0 - z) * n + z * h
    hnew_ref[...] = hnew
    hnew16_ref[...] = hnew.astype(jnp.bfloat16)


def _scatgru(colrow, msg, invdeg, h, convb, wiht, whht, bih, bhh):
    return pl.pallas_call(
        _scatgru_body,
        grid=(N // NB,),
        in_specs=[
            pl.BlockSpec((1, E), lambda i: (0, 0)),
            pl.BlockSpec((E, D), lambda i: (0, 0)),
            pl.BlockSpec((NB, 1), lambda i: (i, 0)),
            pl.BlockSpec((NB, D), lambda i: (i, 0)),
            pl.BlockSpec((1, D), lambda i: (0, 0)),
            pl.BlockSpec((D, 3 * D), lambda i: (0, 0)),
            pl.BlockSpec((D, 3 * D), lambda i: (0, 0)),
            pl.BlockSpec((1, 3 * D), lambda i: (0, 0)),
            pl.BlockSpec((1, 3 * D), lambda i: (0, 0)),
        ],
        out_specs=(pl.BlockSpec((NB, D), lambda i: (i, 0)),
                   pl.BlockSpec((NB, D), lambda i: (i, 0))),
        out_shape=(jax.ShapeDtypeStruct((N, D), jnp.float32),
                   jax.ShapeDtypeStruct((N, D), jnp.bfloat16)),
    )(colrow, msg, invdeg, h, convb, wiht, whht, bih, bhh)


def _readout_body(srcr_ref, srcc_ref, w1rt_ref, w1ct_ref, b1_ref,
                  w2t_ref, b2_ref, w6rt_ref, w6ct_ref, b6a_ref, bng_ref,
                  bnb_ref, w6bt_ref, b6b_ref, o2_ref, precs_ref):
    accr = srcr_ref[...]
    accc = srcc_ref[...]
    o1 = jnp.maximum(
        jnp.dot(accr, w1rt_ref[...], preferred_element_type=jnp.float32,
                precision=_DEF)
        + jnp.dot(accc, w1ct_ref[...], preferred_element_type=jnp.float32,
                  precision=_DEF)
        + b1_ref[...], 0.0)
    o2_ref[...] = jnp.dot(o1, w2t_ref[...], preferred_element_type=jnp.float32,
                          precision=_DEF) + b2_ref[...]
    p = (jnp.dot(accr, w6rt_ref[...], preferred_element_type=jnp.float32,
                 precision=_DEF)
         + jnp.dot(accc, w6ct_ref[...], preferred_element_type=jnp.float32,
                   precision=_DEF)
         + b6a_ref[...])
    mu = jnp.mean(p, axis=0, keepdims=True)
    var = jnp.mean((p - mu) ** 2, axis=0, keepdims=True)
    p = (p - mu) * lax.rsqrt(var + 1e-5) * bng_ref[...] + bnb_ref[...]
    p = jnp.maximum(p, 0.0)
    precs_ref[...] = jnp.dot(p, w6bt_ref[...], preferred_element_type=jnp.float32,
                             precision=_DEF) + b6b_ref[...]


def _readout(srcr, srcc, w1rt, w1ct, b1, w2t, b2, w6rt, w6ct, b6a,
             bng, bnb, w6bt, b6b):
    return pl.pallas_call(
        _readout_body,
        out_shape=(
            jax.ShapeDtypeStruct((E, D), jnp.float32),
            jax.ShapeDtypeStruct((E, 128), jnp.float32),
        ),
    )(srcr, srcc, w1rt, w1ct, b1, w2t, b2, w6rt, w6ct, b6a, bng,
      bnb, w6bt, b6b)


def kernel(x, pos, edge_index, edge_atr, node_emb_table, lin0_W, lin0_b,
           mnn1_W, mnn1_b, mnn2_W, mnn2_b, conv_b, gru_Wih, gru_Whh, gru_bih,
           gru_bhh, lin1_W, lin1_b, lin2_W, lin2_b, lin6a_W, lin6a_b, bn_g,
           bn_b, lin6b_W, lin6b_b):
    row = edge_index[0].astype(jnp.int32)
    col = edge_index[1].astype(jnp.int32)
    col16 = col.reshape(E // 128, 128)
    colrow = col.reshape(1, E)

    w0t = lin0_W.T
    b0 = lin0_b.reshape(1, D)
    m1wt = mnn1_W.T
    m1b = mnn1_b.reshape(1, K128)
    m2t = mnn2_W.T.astype(jnp.bfloat16)
    b2row = mnn2_b.reshape(1, D * D)
    convb = conv_b.reshape(1, D)
    wiht = gru_Wih.T
    whht = gru_Whh.T
    bih = gru_bih.reshape(1, 3 * D)
    bhh = gru_bhh.reshape(1, 3 * D)
    w1rt = lin1_W[:, :D].T
    w1ct = lin1_W[:, D:].T
    b1 = lin1_b.reshape(1, D)
    w2t = jnp.pad(lin2_W.T, ((0, 0), (0, D - 242)))
    b2 = jnp.pad(lin2_b, (0, D - 242)).reshape(1, D)
    w6rt = lin6a_W[:, :D].T
    w6ct = lin6a_W[:, D:].T
    b6a = lin6a_b.reshape(1, 128)
    bng = bn_g.reshape(1, 128)
    bnb = bn_b.reshape(1, 128)
    w6bt = jnp.pad(lin6b_W.T, ((0, 0), (0, 127)))
    b6b = jnp.pad(lin6b_b, (0, 127)).reshape(1, 128)

    out0, hid, invdeg, out016 = _prologue(x.astype(jnp.int32), pos, col16,
                                          node_emb_table, w0t, b0, edge_atr,
                                          m1wt, m1b)
    h, h16 = out0, out016
    row2d = row.reshape(E, 1)
    for _ in range(3):
        msg = _msg(row2d, hid, h16, m2t, b2row)
        h, h16 = _scatgru(colrow, msg, invdeg, h, convb, wiht, whht, bih, bhh)

    src_rc = _sc_gather(h, jnp.concatenate([row, col]), 2 * E)
    o2p, precsp = _readout(src_rc[:E], src_rc[E:], w1rt, w1ct, b1, w2t, b2,
                           w6rt, w6ct, b6a, bng, bnb, w6bt, b6b)
    return (o2p[:, :242], precsp[:, :1])
